# baseline (speedup 1.0000x reference)
import functools

import jax
import jax.numpy as jnp
from jax import lax
from jax.experimental import pallas as pl
from jax.experimental.pallas import tpu as pltpu
from jax.experimental.pallas import tpu_sc as plsc

N = 50000
E = 800000
B = 64
FEAT = 64
NHID = 256
K = 20

SLICE = 32
CH = 128
NSUB = 16
NCORE = 2
EPC = 50688
EPAD = EPC * NSUB
PD = 6
NCHUNK = EPC // CH
NGRP = NCHUNK // PD
NACC = 50048
ZR = 92
RPS = N // NSUB

R = 1000
GRID = N // R

_f32 = jnp.float32
_i32 = jnp.int32


@functools.lru_cache(maxsize=None)
def _make_agg(S):
    mesh = plsc.VectorSubcoreMesh(core_axis_name="c", subcore_axis_name="s")
    rounds = S // NCORE
    scr = (
        [pltpu.VMEM((CH,), _i32) for _ in range(2 * PD)]
        + [pltpu.VMEM((CH, SLICE), _f32) for _ in range(PD)]
        + [pltpu.VMEM((ZR, SLICE), _f32),
           pltpu.VMEM_SHARED((NACC, SLICE), _f32)]
        + [pltpu.SemaphoreType.DMA for _ in range(4 * PD)]
    )

    @functools.partial(
        pl.kernel,
        out_type=jax.ShapeDtypeStruct((N, S * SLICE), _f32),
        mesh=mesh,
        scratch_types=scr,
        compiler_params=pltpu.CompilerParams(
            use_tc_tiling_on_sc=False, needs_layout_passes=False),
    )
    def agg(tbl, srcS, dstp, out, *scrr):
        gidx_v = scrr[0:PD]
        dst_v = scrr[PD:2 * PD]
        rows_v = scrr[2 * PD:3 * PD]
        zero_v = scrr[3 * PD]
        acc = scrr[3 * PD + 1]
        isem = scrr[3 * PD + 2:3 * PD + 2 + PD]
        jsem = scrr[3 * PD + 2 + PD:3 * PD + 2 + 2 * PD]
        gsem = scrr[3 * PD + 2 + 2 * PD:3 * PD + 2 + 3 * PD]
        ssem = scrr[3 * PD + 2 + 3 * PD:3 * PD + 2 + 4 * PD]
        cid = lax.axis_index("c")
        sid = lax.axis_index("s")

        def zb(i, _):
            zero_v[i // 2, pl.ds((i % 2) * 16, 16)] = jnp.zeros((16,), _f32)
            return 0
        lax.fori_loop(0, ZR * 2, zb, 0)

        for b in range(PD):
            pltpu.async_copy(
                srcS.at[pl.ds(sid * EPC + b * CH, CH)], gidx_v[b], isem[b])
            pltpu.async_copy(
                dstp.at[pl.ds(sid * EPC + b * CH, CH)], dst_v[b], jsem[b])

        def one_round(r, _):
            s_dyn = NCORE * r + cid
            for z in range(NACC // (ZR * NSUB)):
                pltpu.sync_copy(
                    zero_v,
                    acc.at[pl.ds(sid * (NACC // NSUB) + z * ZR, ZR)])
            plsc.subcore_barrier()

            def group(gr, _):
                dg = []
                for b in range(PD):
                    pltpu.make_async_copy(
                        srcS.at[pl.ds(0, CH)], gidx_v[b], isem[b]).wait()
                    for j in range(CH // 16):
                        gidx_v[b][pl.ds(j * 16, 16)] = (
                            gidx_v[b][pl.ds(j * 16, 16)] + s_dyn)
                    dg.append(pltpu.async_copy(
                        tbl.at[gidx_v[b]], rows_v[b], gsem[b]))
                ds_ = []
                for b in range(PD):
                    dg[b].wait()
                    pltpu.make_async_copy(
                        dstp.at[pl.ds(0, CH)], dst_v[b], jsem[b]).wait()
                    ds_.append(pltpu.async_copy(
                        rows_v[b], acc.at[dst_v[b]], ssem[b], add=True))
                nxt = jnp.where(gr + 1 < NGRP, gr + 1, 0)
                for b in range(PD):
                    ds_[b].wait()
                    base = sid * EPC + (nxt * PD + b) * CH
                    pltpu.async_copy(
                        srcS.at[pl.ds(base, CH)], gidx_v[b], isem[b])
                    pltpu.async_copy(
                        dstp.at[pl.ds(base, CH)], dst_v[b], jsem[b])
                return 0
            lax.fori_loop(0, NGRP, group, 0)
            plsc.subcore_barrier()
            pltpu.sync_copy(
                acc.at[pl.ds(sid * RPS, RPS)],
                out.at[pl.ds(sid * RPS, RPS),
                       pl.ds(s_dyn * SLICE, SLICE)])
            plsc.subcore_barrier()
            return 0
        lax.fori_loop(0, rounds, one_round, 0)
        for b in range(PD):
            pltpu.make_async_copy(
                srcS.at[pl.ds(0, CH)], gidx_v[b], isem[b]).wait()
            pltpu.make_async_copy(
                dstp.at[pl.ds(0, CH)], dst_v[b], jsem[b]).wait()

    return agg


def _agg2(tbl, srcS, dstp):
    return _make_agg(2)(tbl, srcS, dstp)


def _agg8(tbl, srcS, dstp):
    return _make_agg(8)(tbl, srcS, dstp)


_VPAD = 50016
_BIG = 1 << 30
_NEG = -3.0e38


def _topk_kernel(lc, cv, out, vals, cnts_v, row_v, sem):
    cid = lax.axis_index("c")
    sid = lax.axis_index("s")
    w = sid * NCORE + cid
    pltpu.sync_copy(lc, vals.at[pl.ds(0, N)])
    pltpu.sync_copy(cv, cnts_v)
    lanes = lax.iota(_i32, 16)

    for gi in range(2):
        g = w + 32 * gi
        st_f = jnp.zeros((), _f32)
        c_f = jnp.zeros((), _f32)
        for j in range(B // 16):
            v = cnts_v[pl.ds(j * 16, 16)]
            lane = lanes + (j * 16)
            st_f = st_f + jnp.sum(jnp.where(lane < g, v, 0.0))
            c_f = c_f + jnp.sum(jnp.where(lane == g, v, 0.0))
        start = st_f.astype(_i32)
        cnt = c_f.astype(_i32)
        nv = (cnt + 15) // 16

        def sel_pass(k, _):
            def mx(i, m):
                pos = lanes + i * 16
                v = vals[pl.ds(start + i * 16, 16)]
                v = jnp.where(pos < cnt, v, _NEG)
                return jnp.maximum(m, jnp.max(v))
            m = lax.fori_loop(0, nv, mx, jnp.full((), _NEG, _f32))

            def fp(i, p):
                pos = lanes + i * 16
                v = vals[pl.ds(start + i * 16, 16)]
                hit = (v == m) & (pos < cnt)
                return jnp.minimum(p, jnp.min(jnp.where(hit, pos, _BIG)))
            p = lax.fori_loop(0, nv, fp, jnp.full((), _BIG, _i32))

            valk = jnp.where(k < cnt, p, k).astype(_f32)
            plsc.store_scatter(
                row_v, [jnp.broadcast_to(k, (16,))],
                jnp.broadcast_to(valk, (16,)),
                mask=lanes == 0)
            plsc.store_scatter(
                vals, [jnp.broadcast_to(start + p, (16,))],
                jnp.full((16,), _NEG, _f32),
                mask=lanes == jnp.where(k < cnt, 0, 999))
            return 0
        lax.fori_loop(0, K, sel_pass, 0)
        pltpu.sync_copy(row_v, out.at[g])


@functools.lru_cache(maxsize=None)
def _make_topk():
    return functools.partial(
        pl.kernel,
        out_type=jax.ShapeDtypeStruct((B, 32), _f32),
        mesh=plsc.VectorSubcoreMesh(core_axis_name="c", subcore_axis_name="s"),
        scratch_types=[
            pltpu.VMEM((_VPAD,), _f32),
            pltpu.VMEM((B,), _f32),
            pltpu.VMEM((32,), _f32),
            pltpu.SemaphoreType.DMA,
        ],
        compiler_params=pltpu.CompilerParams(
            use_tc_tiling_on_sc=False, needs_layout_passes=False),
    )(_topk_kernel)


def _topk(lc, cv):
    return _make_topk()(lc, cv)


def _lin_body(x_ref, a_ref, w_ref, b_ref, h_ref, st_ref):
    i = pl.program_id(0)
    z = x_ref[...] + a_ref[...]
    h = jnp.dot(z, w_ref[...], preferred_element_type=_f32) + b_ref[...]
    h = jnp.maximum(h, 0.0)
    h_ref[...] = h

    @pl.when(i == 0)
    def _():
        st_ref[...] = jnp.zeros_like(st_ref)
        st_ref[2:3, :] = jnp.sum(h, axis=0, keepdims=True) / R
    d = h - st_ref[2:3, :]
    st_ref[0:1, :] += jnp.sum(h, axis=0, keepdims=True)
    st_ref[1:2, :] += jnp.sum(d * d, axis=0, keepdims=True)


def _lin(x, a, w, b):
    fi = x.shape[1]
    return pl.pallas_call(
        _lin_body,
        grid=(GRID,),
        in_specs=[
            pl.BlockSpec((R, fi), lambda i: (i, 0)),
            pl.BlockSpec((R, fi), lambda i: (i, 0)),
            pl.BlockSpec((fi, NHID), lambda i: (0, 0)),
            pl.BlockSpec((1, NHID), lambda i: (0, 0)),
        ],
        out_specs=[
            pl.BlockSpec((R, NHID), lambda i: (i, 0)),
            pl.BlockSpec((8, NHID), lambda i: (0, 0)),
        ],
        out_shape=[
            jax.ShapeDtypeStruct((N, NHID), _f32),
            jax.ShapeDtypeStruct((8, NHID), _f32),
        ],
    )(x, a, w, b.reshape(1, NHID))


def _bn_body(h_ref, st_ref, g_ref, be_ref, o_ref):
    mu = st_ref[0:1, :] / N
    dm = mu - st_ref[2:3, :]
    var = st_ref[1:2, :] / N - dm * dm
    o_ref[...] = ((h_ref[...] - mu) * lax.rsqrt(var + 1e-5)
                  * g_ref[...] + be_ref[...])


def _bn(h, st, g, be):
    return pl.pallas_call(
        _bn_body,
        grid=(GRID,),
        in_specs=[
            pl.BlockSpec((R, NHID), lambda i: (i, 0)),
            pl.BlockSpec((8, NHID), lambda i: (0, 0)),
            pl.BlockSpec((1, NHID), lambda i: (0, 0)),
            pl.BlockSpec((1, NHID), lambda i: (0, 0)),
        ],
        out_specs=pl.BlockSpec((R, NHID), lambda i: (i, 0)),
        out_shape=jax.ShapeDtypeStruct((N, NHID), _f32),
    )(h, st, g.reshape(1, NHID), be.reshape(1, NHID))


def _pool_body(x2_ref, a_ref, w_ref, b_ref, x1_ref, bat_ref,
               pe_ref, lc_ref):
    i = pl.program_id(0)
    z = x2_ref[...] + a_ref[...]
    h3 = jnp.maximum(
        jnp.dot(z, w_ref[...], preferred_element_type=_f32) + b_ref[...], 0.0)
    xj = jnp.maximum(jnp.maximum(x1_ref[...], x2_ref[...]), h3)
    oh = (lax.broadcasted_iota(_i32, (B, R), 0)
          == bat_ref[0]).astype(_f32)
    xje = jnp.concatenate([xj, jnp.ones((R, 128), _f32)], axis=1)

    @pl.when(i == 0)
    def _():
        pe_ref[...] = jnp.zeros_like(pe_ref)
    pe_ref[...] += jnp.dot(oh, xje, preferred_element_type=_f32)
    lc_ref[...] = jnp.broadcast_to(xj[:, 255:256], (R, 128))


def _pool(x2, a2, w3, b3, x1, batch3):
    return pl.pallas_call(
        _pool_body,
        grid=(GRID,),
        in_specs=[
            pl.BlockSpec((R, NHID), lambda i: (i, 0)),
            pl.BlockSpec((R, NHID), lambda i: (i, 0)),
            pl.BlockSpec((NHID, NHID), lambda i: (0, 0)),
            pl.BlockSpec((1, NHID), lambda i: (0, 0)),
            pl.BlockSpec((R, NHID), lambda i: (i, 0)),
            pl.BlockSpec((1, 1, R), lambda i: (i, 0, 0)),
        ],
        out_specs=[
            pl.BlockSpec((B, NHID + 128), lambda i: (0, 0)),
            pl.BlockSpec((R, 128), lambda i: (i, 0)),
        ],
        out_shape=[
            jax.ShapeDtypeStruct((B, NHID + 128), _f32),
            jax.ShapeDtypeStruct((N, 128), _f32),
        ],
    )(x2, a2, w3, b3.reshape(1, NHID), x1, batch3)


def _head_body(p_ref, w1_ref, b1_ref, w2_ref, b2_ref, w3_ref, b3_ref, o_ref):
    t = jnp.dot(p_ref[...], w1_ref[...], preferred_element_type=_f32)
    t = jnp.maximum(t + b1_ref[...], 0.0)
    t = jnp.dot(t, w2_ref[...], preferred_element_type=_f32)
    t = jnp.maximum(t + b2_ref[...], 0.0)
    t = jnp.dot(t, w3_ref[...], preferred_element_type=_f32) + b3_ref[...]
    m = jnp.max(t, axis=1, keepdims=True)
    lse = jnp.log(jnp.sum(jnp.exp(t - m), axis=1, keepdims=True)) + m
    o_ref[...] = t - lse


def _head(pooled, wf1, bf1, wf2, bf2, wf3, bf3):
    h1, h2 = NHID // 2, NHID // 4
    return pl.pallas_call(
        _head_body,
        in_specs=[
            pl.BlockSpec((B, NHID), lambda: (0, 0)),
            pl.BlockSpec((NHID, h1), lambda: (0, 0)),
            pl.BlockSpec((1, h1), lambda: (0, 0)),
            pl.BlockSpec((h1, h2), lambda: (0, 0)),
            pl.BlockSpec((1, h2), lambda: (0, 0)),
            pl.BlockSpec((h2, 2), lambda: (0, 0)),
            pl.BlockSpec((1, 2), lambda: (0, 0)),
        ],
        out_specs=pl.BlockSpec((B, 2), lambda: (0, 0)),
        out_shape=jax.ShapeDtypeStruct((B, 2), _f32),
    )(pooled, wf1, bf1.reshape(1, h1), wf2, bf2.reshape(1, h2),
      wf3, bf3.reshape(1, 2))


def kernel(x, edge_index, batch, W1, b1, g1, be1, W2, b2, g2, be2, W3, b3,
           Wf1, bf1, Wf2, bf2, Wf3, bf3):
    src = edge_index[0]
    dst = edge_index[1]
    pad = EPAD - E
    zpad = jnp.zeros((pad,), _i32)
    dstp = jnp.concatenate([dst, N + (jnp.arange(pad, dtype=_i32) % 16)])
    src2 = jnp.concatenate([src * 2, zpad])
    src8 = jnp.concatenate([src * 8, zpad])
    batch3 = batch.reshape(GRID, 1, R)

    a0 = _agg2(x.reshape(N * 2, SLICE), src2, dstp)
    h1, st1 = _lin(x, a0, W1, b1)
    x1 = _bn(h1, st1, g1, be1)

    a1 = _agg8(x1.reshape(N * 8, SLICE), src8, dstp)
    h2, st2 = _lin(x1, a1, W2, b2)
    x2 = _bn(h2, st2, g2, be2)

    a2 = _agg8(x2.reshape(N * 8, SLICE), src8, dstp)
    pe, lc = _pool(x2, a2, W3, b3, x1, batch3)

    pooled = pe[:, :NHID]
    counts = pe[:, NHID]
    sel = _topk(lc[:, 0], counts)[:, :K]
    logp = _head(pooled, Wf1, bf1, Wf2, bf2, Wf3, bf3)
    return (logp, sel)
